# Initial kernel scaffold; baseline (speedup 1.0000x reference)
import jax
import jax.numpy as jnp
from jax.experimental import pallas as pl


def kernel(x, edge_index, edge_attr, u, batch, params):
    raise NotImplementedError("write your pallas kernel here")


# speedup vs baseline: 3.4142x; 3.4142x over previous
import dataclasses
import functools

import jax
import jax.numpy as jnp
from jax import lax
from jax.experimental import pallas as pl
from jax.experimental.pallas import tpu as pltpu
from jax.experimental.pallas import tpu_sc as plsc

N = 10000
E = 320000
D = 128
EA = 16
EPS = 1e-5

NC = 2
NS = 16
NW = NC * NS
EPW = E // NW
CH = 80
KJ = EPW // CH

BE = 4000
HI = lax.Precision.HIGHEST
F32 = jnp.float32


def _sc_compiler_params():
    cp = pltpu.CompilerParams()
    if "needs_layout_passes" in pltpu.CompilerParams.__dataclass_fields__:
        cp = dataclasses.replace(cp, needs_layout_passes=False)
    return cp


def _zero_hist(hist_v):
    z = jnp.zeros((16,), F32)

    @pl.loop(0, N, step=16)
    def _(i):
        hist_v[pl.ds(i, 16)] = z


def _hist_add(hist_v, idx_v, j):
    one = jnp.ones((16,), F32)
    for k in range(CH // 16):
        iv = idx_v[j, pl.ds(k * 16, 16)]
        plsc.addupdate_scatter(hist_v, [iv], one)


def _sc_gather_body(x_hbm, row3_hbm, xe_hbm, degp_hbm,
                    idx_v, rows_v, hist_v, sem):
    cid = lax.axis_index("c")
    sid = lax.axis_index("s")
    wid = sid * NC + cid

    pltpu.sync_copy(row3_hbm.at[wid], idx_v)
    _zero_hist(hist_v)
    base = wid * EPW

    @pl.loop(0, KJ)
    def _(j):
        pltpu.async_copy(x_hbm.at[idx_v.at[j]], rows_v, sem).wait()
        pltpu.sync_copy(rows_v, xe_hbm.at[pl.ds(base + j * CH, CH)])
        _hist_add(hist_v, idx_v, j)

    pltpu.sync_copy(hist_v, degp_hbm.at[wid])


def _sc_gather(x, row3):
    mesh = plsc.VectorSubcoreMesh(core_axis_name="c", subcore_axis_name="s")
    f = pl.kernel(
        _sc_gather_body,
        mesh=mesh,
        out_type=[jax.ShapeDtypeStruct((E, D), F32),
                  jax.ShapeDtypeStruct((NW, N), F32)],
        scratch_types=[pltpu.VMEM((KJ, CH), jnp.int32),
                       pltpu.VMEM((CH, D), F32),
                       pltpu.VMEM((N,), F32),
                       pltpu.SemaphoreType.DMA],
        compiler_params=_sc_compiler_params(),
    )
    return f(x, row3)


def _sc_scatter_body(a2_hbm, col3_hbm, z128_hbm, sp_hbm, cntp_hbm,
                     idx_v, idx1_v, data_v, hist_v, acc_sh, sem):
    cid = lax.axis_index("c")
    sid = lax.axis_index("s")
    wid = sid * NC + cid

    @pl.when(sid == 0)
    def _():
        pltpu.sync_copy(z128_hbm, acc_sh)

    plsc.subcore_barrier()

    pltpu.sync_copy(col3_hbm.at[wid], idx_v)
    _zero_hist(hist_v)
    base = wid * EPW

    @pl.loop(0, KJ)
    def _(j):
        for k in range(CH // 16):
            idx1_v[pl.ds(k * 16, 16)] = idx_v[j, pl.ds(k * 16, 16)]
        pltpu.async_copy(a2_hbm.at[pl.ds(base + j * CH, CH)], data_v,
                         sem).wait()
        pltpu.sync_copy(data_v, acc_sh.at[idx1_v], add=True)
        _hist_add(hist_v, idx_v, j)

    pltpu.sync_copy(hist_v, cntp_hbm.at[wid])
    plsc.subcore_barrier()

    @pl.when(sid < 10)
    def _():
        r0 = sid * 1000
        pltpu.sync_copy(acc_sh.at[pl.ds(r0, 1000)],
                        sp_hbm.at[pl.ds(cid * N + r0, 1000)])


def _sc_scatter(a2, col3, z128):
    mesh = plsc.VectorSubcoreMesh(core_axis_name="c", subcore_axis_name="s")
    f = pl.kernel(
        _sc_scatter_body,
        mesh=mesh,
        out_type=[jax.ShapeDtypeStruct((NC * N, D), F32),
                  jax.ShapeDtypeStruct((NW, N), F32)],
        scratch_types=[pltpu.VMEM((KJ, CH), jnp.int32),
                       pltpu.VMEM((CH,), jnp.int32),
                       pltpu.VMEM((CH, D), F32),
                       pltpu.VMEM((N,), F32),
                       pltpu.VMEM_SHARED((N, D), F32),
                       pltpu.SemaphoreType.DMA],
        compiler_params=_sc_compiler_params(),
    )
    return f(a2, col3, z128)


def _ea_stats_body(ea_ref, o_ref):
    i = pl.program_id(0)

    @pl.when(i == 0)
    def _():
        o_ref[...] = jnp.zeros_like(o_ref)

    blk = ea_ref[...]
    s = jnp.sum(blk, axis=0, keepdims=True)
    sq = jnp.sum(blk * blk, axis=0, keepdims=True)
    o_ref[...] += jnp.concatenate([s, sq], axis=0)


def _prep_body(x_ref, deg_ref, o_ref):
    xb = x_ref[...]
    deg = deg_ref[...]
    wx = xb * deg
    r0 = jnp.sum(wx, axis=0, keepdims=True)
    r1 = jnp.sum(wx * xb, axis=0, keepdims=True)
    r2 = jnp.sum(xb, axis=0, keepdims=True)
    r3 = jnp.sum(xb * xb, axis=0, keepdims=True)
    o_ref[...] = jnp.concatenate([r0, r1, r2, r3], axis=0)


def _p2_body(xe_ref, ea_ref, w0x_ref, w0e_ref, b0_ref, a1_ref, st_ref):
    i = pl.program_id(0)

    @pl.when(i == 0)
    def _():
        st_ref[...] = jnp.zeros_like(st_ref)

    acc = jnp.dot(xe_ref[...].astype(jnp.bfloat16), w0x_ref[...],
                  preferred_element_type=F32)
    acc += jnp.dot(ea_ref[...].astype(jnp.bfloat16), w0e_ref[...],
                   preferred_element_type=F32)
    a1 = jnp.maximum(acc + b0_ref[...], 0.0)
    a1_ref[...] = a1.astype(jnp.bfloat16)
    s = jnp.sum(a1, axis=0, keepdims=True)
    sq = jnp.sum(a1 * a1, axis=0, keepdims=True)
    st_ref[...] += jnp.concatenate([s, sq], axis=0)


def _p3_body(a1_ref, w1_ref, b1_ref, a2_ref, st_ref):
    i = pl.program_id(0)

    @pl.when(i == 0)
    def _():
        st_ref[...] = jnp.zeros_like(st_ref)

    a2 = jnp.maximum(
        jnp.dot(a1_ref[...], w1_ref[...], preferred_element_type=F32)
        + b1_ref[...], 0.0)
    a2_ref[...] = a2
    s = jnp.sum(a2, axis=0, keepdims=True)
    sq = jnp.sum(a2 * a2, axis=0, keepdims=True)
    st_ref[...] += jnp.concatenate([s, sq], axis=0)


def _k5_body(sp_ref, cnt_ref, w2_ref, b2_ref, mean_ref, stm_ref):
    sp = sp_ref[...]
    s = sp[0] + sp[1]
    cnt = cnt_ref[...]
    ssum = jnp.dot(s, w2_ref[...], preferred_element_type=F32,
                   precision=HI) + cnt * b2_ref[...]
    mean = ssum / jnp.maximum(cnt, 1.0)
    mean_ref[...] = mean
    ms = jnp.sum(mean, axis=0, keepdims=True)
    msq = jnp.sum(mean * mean, axis=0, keepdims=True)
    stm_ref[...] = jnp.concatenate([ms, msq], axis=0)


def _k6_body(x_ref, mean_ref, w20x_ref, w20m_ref, b20_ref,
             w21_ref, b21_ref, g21_ref, be21_ref,
             w22_ref, b22_ref, g22_ref, be22_ref, o_ref):
    a = jnp.dot(x_ref[...], w20x_ref[...], preferred_element_type=F32,
                precision=HI)
    a += jnp.dot(mean_ref[...], w20m_ref[...], preferred_element_type=F32,
                 precision=HI)
    a = jnp.maximum(a + b20_ref[...], 0.0)

    m = jnp.mean(a, axis=0, keepdims=True)
    v = jnp.mean((a - m) ** 2, axis=0, keepdims=True)
    hn = (a - m) / jnp.sqrt(v + EPS) * g21_ref[...] + be21_ref[...]
    a = jnp.maximum(
        jnp.dot(hn, w21_ref[...], preferred_element_type=F32, precision=HI)
        + b21_ref[...], 0.0)

    m = jnp.mean(a, axis=0, keepdims=True)
    v = jnp.mean((a - m) ** 2, axis=0, keepdims=True)
    hn = (a - m) / jnp.sqrt(v + EPS) * g22_ref[...] + be22_ref[...]
    o_ref[...] = (
        jnp.dot(hn, w22_ref[...], preferred_element_type=F32, precision=HI)
        + b22_ref[...])


def _full(shape):
    return pl.BlockSpec(shape, lambda i: tuple(0 for _ in shape))


def _ea_stats(ea):
    grid = (E // 8000,)
    return pl.pallas_call(
        _ea_stats_body,
        grid=grid,
        in_specs=[pl.BlockSpec((8000, EA), lambda i: (i, 0))],
        out_specs=pl.BlockSpec((2, EA), lambda i: (0, 0)),
        out_shape=jax.ShapeDtypeStruct((2, EA), F32),
    )(ea)


def _prep(x, deg_col):
    return pl.pallas_call(
        _prep_body,
        in_specs=[pl.BlockSpec((N, D), lambda: (0, 0)),
                  pl.BlockSpec((N, 1), lambda: (0, 0))],
        out_specs=pl.BlockSpec((4, D), lambda: (0, 0)),
        out_shape=jax.ShapeDtypeStruct((4, D), F32),
    )(x, deg_col)


def _p2(xe, ea, w0x, w0e, b0):
    grid = (E // BE,)
    return pl.pallas_call(
        _p2_body,
        grid=grid,
        in_specs=[pl.BlockSpec((BE, D), lambda i: (i, 0)),
                  pl.BlockSpec((BE, EA), lambda i: (i, 0)),
                  pl.BlockSpec((D, D), lambda i: (0, 0)),
                  pl.BlockSpec((EA, D), lambda i: (0, 0)),
                  pl.BlockSpec((1, D), lambda i: (0, 0))],
        out_specs=[pl.BlockSpec((BE, D), lambda i: (i, 0)),
                   pl.BlockSpec((2, D), lambda i: (0, 0))],
        out_shape=[jax.ShapeDtypeStruct((E, D), jnp.bfloat16),
                   jax.ShapeDtypeStruct((2, D), F32)],
    )(xe, ea, w0x, w0e, b0)


def _p3(a1, w1, b1):
    grid = (E // BE,)
    return pl.pallas_call(
        _p3_body,
        grid=grid,
        in_specs=[pl.BlockSpec((BE, D), lambda i: (i, 0)),
                  pl.BlockSpec((D, D), lambda i: (0, 0)),
                  pl.BlockSpec((1, D), lambda i: (0, 0))],
        out_specs=[pl.BlockSpec((BE, D), lambda i: (i, 0)),
                   pl.BlockSpec((2, D), lambda i: (0, 0))],
        out_shape=[jax.ShapeDtypeStruct((E, D), F32),
                   jax.ShapeDtypeStruct((2, D), F32)],
    )(a1, w1, b1)


def _k5(sp, cnt_col, w2, b2):
    return pl.pallas_call(
        _k5_body,
        in_specs=[pl.BlockSpec((NC, N, D), lambda: (0, 0, 0)),
                  pl.BlockSpec((N, 1), lambda: (0, 0)),
                  pl.BlockSpec((D, D), lambda: (0, 0)),
                  pl.BlockSpec((1, D), lambda: (0, 0))],
        out_specs=[pl.BlockSpec((N, D), lambda: (0, 0)),
                   pl.BlockSpec((2, D), lambda: (0, 0))],
        out_shape=[jax.ShapeDtypeStruct((N, D), F32),
                   jax.ShapeDtypeStruct((2, D), F32)],
    )(sp, cnt_col, w2, b2)


def _k6(x, mean, w20x, w20m, b20, w21, b21, g21, be21, w22, b22, g22, be22):
    vec = pl.BlockSpec((1, D), lambda: (0, 0))
    mat = pl.BlockSpec((D, D), lambda: (0, 0))
    return pl.pallas_call(
        _k6_body,
        in_specs=[pl.BlockSpec((N, D), lambda: (0, 0)),
                  pl.BlockSpec((N, D), lambda: (0, 0)),
                  mat, mat, vec, mat, vec, vec, vec, mat, vec, vec, vec],
        out_specs=pl.BlockSpec((N, D), lambda: (0, 0)),
        out_shape=jax.ShapeDtypeStruct((N, D), F32),
    )(x, mean, w20x, w20m, b20, w21, b21, g21, be21, w22, b22, g22, be22)


def _fold(m, v, g, be, W, b):
    s = g / jnp.sqrt(v + EPS)
    t = be - m * s
    return s[:, None] * W, (t @ W + b)[None, :]


def kernel(x, edge_index, edge_attr, u, batch, params):
    del u, batch
    p = params
    row3 = edge_index[0].reshape(NW, KJ, CH)
    col3 = edge_index[1].reshape(NW, KJ, CH)
    z128 = jnp.zeros((N, D), F32)

    xe, degp = _sc_gather(x, row3)
    deg_col = jnp.sum(degp, axis=0)[:, None]

    east = _ea_stats(edge_attr)
    pst = _prep(x, deg_col)

    m0 = jnp.concatenate([pst[0] / E, east[0] / E])
    v0 = jnp.concatenate([pst[1] / E - (pst[0] / E) ** 2,
                          east[1] / E - (east[0] / E) ** 2])
    W0f, b0f = _fold(m0, v0, p['m1_g0'], p['m1_be0'], p['m1_W0'], p['m1_b0'])
    w0x = W0f[:D].astype(jnp.bfloat16)
    w0e = W0f[D:].astype(jnp.bfloat16)

    a1, st1 = _p2(xe, edge_attr, w0x, w0e, b0f)

    m1 = st1[0] / E
    v1 = st1[1] / E - m1 * m1
    W1f, b1f = _fold(m1, v1, p['m1_g1'], p['m1_be1'], p['m1_W1'], p['m1_b1'])
    a2, st2 = _p3(a1, W1f.astype(jnp.bfloat16), b1f)

    m2 = st2[0] / E
    v2 = st2[1] / E - m2 * m2
    W2f, b2f = _fold(m2, v2, p['m1_g2'], p['m1_be2'], p['m1_W2'], p['m1_b2'])

    sp, cntp = _sc_scatter(a2, col3, z128)
    sp = sp.reshape(NC, N, D)
    cnt_col = jnp.sum(cntp, axis=0)[:, None]

    mean, stm = _k5(sp, cnt_col, W2f, b2f)

    mh = jnp.concatenate([pst[2] / N, stm[0] / N])
    vh = jnp.concatenate([pst[3] / N - (pst[2] / N) ** 2,
                          stm[1] / N - (stm[0] / N) ** 2])
    W20f, b20f = _fold(mh, vh, p['m2_g0'], p['m2_be0'], p['m2_W0'],
                       p['m2_b0'])

    return _k6(x, mean, W20f[:D], W20f[D:], b20f,
               p['m2_W1'], p['m2_b1'][None, :], p['m2_g1'][None, :],
               p['m2_be1'][None, :],
               p['m2_W2'], p['m2_b2'][None, :], p['m2_g2'][None, :],
               p['m2_be2'][None, :])
